# baseline (speedup 1.0000x reference)
import functools

import jax
import jax.numpy as jnp
from jax import lax
from jax.experimental import pallas as pl
from jax.experimental.pallas import tpu as pltpu
from jax.experimental.pallas import tpu_sc as plsc

_K = 8192
_D = 256
_BR = 96
_NCH = 6


_LOG2E = 1.4426950408889634


def _vq_body(zb_ref, pb_ref, et_ref, prob_ref, pos_prob_ref, idx_ref,
             eel_ref, iota_ref, zt_ref, pt_ref):
    i = pl.program_id(0)
    j = pl.program_id(1)
    @pl.when((i == 0) & (j == 0))
    def _():
        et = et_ref[...]
        ee = jnp.sum(et * et, axis=0, keepdims=True)
        eel_ref[...] = ee * jnp.float32(_LOG2E)
        iota_ref[...] = lax.broadcasted_iota(
            jnp.int32, (1, _K), 1).astype(jnp.float32)

    @pl.when(j == 0)
    def _():
        zt_ref[...] = jnp.transpose(zb_ref[0], (1, 0)).reshape(_NCH, _BR, _D)
        pt_ref[...] = jnp.transpose(pb_ref[0], (1, 0)).reshape(_NCH, _BR, _D)

    eel = eel_ref[...]
    c1 = jnp.float32(2.0 * _LOG2E)

    def soft(mm):
        ex = jnp.exp2(mm * c1 - eel)
        s = jnp.sum(ex, axis=1, keepdims=True)
        return ex * (1.0 / s)

    zb = zt_ref[j]
    mm = jnp.dot(zb, et_ref[...], preferred_element_type=jnp.float32)
    prob_ref[...] = soft(mm)

    zz2 = 0.5 * jnp.sum(zb * zb, axis=1, keepdims=True)
    e = mm - zz2
    mneg = jnp.max(e, axis=1, keepdims=True)
    idxf = jnp.min(jnp.where(e == mneg, iota_ref[...], jnp.float32(_K)),
                   axis=1, keepdims=True)
    idx_ref[...] = idxf.astype(jnp.int32)

    pmm = jnp.dot(pt_ref[j], et_ref[...], preferred_element_type=jnp.float32)
    pos_prob_ref[...] = soft(pmm)


def _distance_softmax(z3, zpos3, et):
    nbatch, _, hw = z3.shape
    n = nbatch * hw
    grid = (nbatch, _NCH)
    return pl.pallas_call(
        _vq_body,
        grid=grid,
        in_specs=[
            pl.BlockSpec((1, _D, hw), lambda i, j: (i, 0, 0)),
            pl.BlockSpec((1, _D, hw), lambda i, j: (i, 0, 0)),
            pl.BlockSpec((_D, _K), lambda i, j: (0, 0)),
        ],
        out_specs=[
            pl.BlockSpec((_BR, _K), lambda i, j: (i * _NCH + j, 0)),
            pl.BlockSpec((_BR, _K), lambda i, j: (i * _NCH + j, 0)),
            pl.BlockSpec((_BR, 1), lambda i, j: (i * _NCH + j, 0)),
        ],
        out_shape=[
            jax.ShapeDtypeStruct((n, _K), jnp.float32),
            jax.ShapeDtypeStruct((n, _K), jnp.float32),
            jax.ShapeDtypeStruct((n, 1), jnp.int32),
        ],
        scratch_shapes=[pltpu.VMEM((1, _K), jnp.float32),
                        pltpu.VMEM((1, _K), jnp.float32),
                        pltpu.VMEM((_NCH, _BR, _D), jnp.float32),
                        pltpu.VMEM((_NCH, _BR, _D), jnp.float32)],
        compiler_params=pltpu.CompilerParams(
            dimension_semantics=("arbitrary", "arbitrary"),
        ),
    )(z3, zpos3, et)


def _sc_gather(table, idx):
    info = plsc.get_sparse_core_info()
    nc, ns = info.num_cores, info.num_subcores
    nw = nc * ns
    b = idx.shape[0]
    b_per_w = b // nw
    chunk = 96
    nch = b_per_w // chunk
    mesh = plsc.VectorSubcoreMesh(core_axis_name="c", subcore_axis_name="s")

    @functools.partial(
        pl.kernel,
        out_type=jax.ShapeDtypeStruct((b, _D), jnp.float32),
        mesh=mesh,
        scratch_types=[
            pltpu.VMEM((nch, chunk), jnp.int32),
            pltpu.VMEM((nch, chunk, _D), jnp.float32),
            pltpu.SemaphoreType.DMA,
        ],
    )
    def k(table_hbm, idx_hbm, out_hbm, idx_v, rows_v, sem):
        wid = lax.axis_index("s") * nc + lax.axis_index("c")
        base = wid * b_per_w
        for c in range(nch):
            pltpu.sync_copy(idx_hbm.at[pl.ds(base + c * chunk, chunk)],
                            idx_v.at[c])
        copies = [
            pltpu.async_copy(table_hbm.at[idx_v.at[c]], rows_v.at[c], sem)
            for c in range(nch)
        ]
        for c in range(nch):
            copies[c].wait()
            pltpu.sync_copy(rows_v.at[c],
                            out_hbm.at[pl.ds(base + c * chunk, chunk)])

    return k(table, idx)


def kernel(z, z_pos, embedding):
    b, d, h, w = z.shape
    z3 = z.reshape(b, _D, h * w)
    zpos3 = z_pos.reshape(b, _D, h * w)
    et = embedding.T

    prob, pos_prob, idx2d = _distance_softmax(z3, zpos3, et)
    idx = idx2d.reshape(-1)

    zq_flat = _sc_gather(embedding, idx)
    z_q = jnp.transpose(zq_flat.reshape(b, h, w, _D), (0, 3, 1, 2))
    return (z_q, prob, pos_prob, idx)


# speedup vs baseline: 1.1279x; 1.1279x over previous
import functools

import jax
import jax.numpy as jnp
from jax import lax
from jax.experimental import pallas as pl
from jax.experimental.pallas import tpu as pltpu
from jax.experimental.pallas import tpu_sc as plsc

_K = 8192
_D = 256
_BR = 128


_LOG2E = 1.4426950408889634


def _vq_body(zb_ref, pb_ref, et_ref, prob_ref, pos_prob_ref, idx_ref,
             eel_ref, iota_ref):
    @pl.when(pl.program_id(0) == 0)
    def _():
        et = et_ref[...]
        ee = jnp.sum(et * et, axis=0, keepdims=True)
        eel_ref[...] = ee * jnp.float32(_LOG2E)
        iota_ref[...] = lax.broadcasted_iota(
            jnp.int32, (1, _K), 1).astype(jnp.float32)

    eel = eel_ref[...]
    c1 = jnp.float32(2.0 * _LOG2E)

    def soft(mm):
        ex = jnp.exp2(mm * c1 - eel)
        s = jnp.sum(ex, axis=1, keepdims=True)
        return ex * (1.0 / s)

    zb = zb_ref[...]
    mm = jnp.dot(zb, et_ref[...], preferred_element_type=jnp.float32)
    prob_ref[...] = soft(mm)

    zz2 = 0.5 * jnp.sum(zb * zb, axis=1, keepdims=True)
    e = mm - zz2
    mneg = jnp.max(e, axis=1, keepdims=True)
    idxf = jnp.min(jnp.where(e == mneg, iota_ref[...], jnp.float32(_K)),
                   axis=1, keepdims=True)
    idx_ref[...] = idxf.astype(jnp.int32).reshape(_BR)

    pmm = jnp.dot(pb_ref[...], et_ref[...], preferred_element_type=jnp.float32)
    pos_prob_ref[...] = soft(pmm)


def _distance_softmax(z_flat, zpos_flat, et):
    n = z_flat.shape[0]
    nb = n // _BR
    grid = (nb,)
    return pl.pallas_call(
        _vq_body,
        grid=grid,
        in_specs=[
            pl.BlockSpec((_BR, _D), lambda i: (i, 0)),
            pl.BlockSpec((_BR, _D), lambda i: (i, 0)),
            pl.BlockSpec((_D, _K), lambda i: (0, 0)),
        ],
        out_specs=[
            pl.BlockSpec((_BR, _K), lambda i: (i, 0)),
            pl.BlockSpec((_BR, _K), lambda i: (i, 0)),
            pl.BlockSpec((_BR,), lambda i: (i,)),
        ],
        out_shape=[
            jax.ShapeDtypeStruct((n, _K), jnp.float32),
            jax.ShapeDtypeStruct((n, _K), jnp.float32),
            jax.ShapeDtypeStruct((n,), jnp.int32),
        ],
        scratch_shapes=[pltpu.VMEM((1, _K), jnp.float32),
                        pltpu.VMEM((1, _K), jnp.float32)],
        compiler_params=pltpu.CompilerParams(
            dimension_semantics=("arbitrary",),
        ),
    )(z_flat, zpos_flat, et)


def _sc_gather(table, idx):
    info = plsc.get_sparse_core_info()
    nc, ns = info.num_cores, info.num_subcores
    nw = nc * ns
    b = idx.shape[0]
    b_per_w = b // nw
    chunk = 96
    nch = b_per_w // chunk
    mesh = plsc.VectorSubcoreMesh(core_axis_name="c", subcore_axis_name="s")

    @functools.partial(
        pl.kernel,
        out_type=jax.ShapeDtypeStruct((b, _D), jnp.float32),
        mesh=mesh,
        scratch_types=[
            pltpu.VMEM((nch, chunk), jnp.int32),
            pltpu.VMEM((nch, chunk, _D), jnp.float32),
            pltpu.SemaphoreType.DMA,
        ],
    )
    def k(table_hbm, idx_hbm, out_hbm, idx_v, rows_v, sem):
        wid = lax.axis_index("s") * nc + lax.axis_index("c")
        base = wid * b_per_w
        for c in range(nch):
            pltpu.sync_copy(idx_hbm.at[pl.ds(base + c * chunk, chunk)],
                            idx_v.at[c])
        copies = [
            pltpu.async_copy(table_hbm.at[idx_v.at[c]], rows_v.at[c], sem)
            for c in range(nch)
        ]
        for c in range(nch):
            copies[c].wait()
            pltpu.sync_copy(rows_v.at[c],
                            out_hbm.at[pl.ds(base + c * chunk, chunk)])

    return k(table, idx)


def kernel(z, z_pos, embedding):
    b, d, h, w = z.shape
    z_flat = jnp.transpose(z, (0, 2, 3, 1)).reshape(-1, _D)
    zpos_flat = jnp.transpose(z_pos, (0, 2, 3, 1)).reshape(-1, _D)
    et = embedding.T

    prob, pos_prob, idx = _distance_softmax(z_flat, zpos_flat, et)

    zq_flat = _sc_gather(embedding, idx)
    z_q = jnp.transpose(zq_flat.reshape(b, h, w, _D), (0, 3, 1, 2))
    return (z_q, prob, pos_prob, idx)


# speedup vs baseline: 1.2387x; 1.0982x over previous
import functools

import jax
import jax.numpy as jnp
from jax import lax
from jax.experimental import pallas as pl
from jax.experimental.pallas import tpu as pltpu
from jax.experimental.pallas import tpu_sc as plsc

_K = 8192
_D = 256
_BR = 256


_LOG2E = 1.4426950408889634


def _vq_body(zb_ref, pb_ref, et_ref, prob_ref, pos_prob_ref, idx_ref,
             eel_ref, iota_ref):
    @pl.when(pl.program_id(0) == 0)
    def _():
        et = et_ref[...]
        ee = jnp.sum(et * et, axis=0, keepdims=True)
        eel_ref[...] = ee * jnp.float32(_LOG2E)
        iota_ref[...] = lax.broadcasted_iota(
            jnp.int32, (1, _K), 1).astype(jnp.float32)

    eel = eel_ref[...]
    c1 = jnp.float32(2.0 * _LOG2E)

    def soft(mm):
        ex = jnp.exp2(mm * c1 - eel)
        s = jnp.sum(ex, axis=1, keepdims=True)
        return ex * (1.0 / s)

    zb = zb_ref[...]
    mm = jnp.dot(zb, et_ref[...], preferred_element_type=jnp.float32)
    prob_ref[...] = soft(mm)

    zz2 = 0.5 * jnp.sum(zb * zb, axis=1, keepdims=True)
    e = mm - zz2
    mneg = jnp.max(e, axis=1, keepdims=True)
    idxf = jnp.min(jnp.where(e == mneg, iota_ref[...], jnp.float32(_K)),
                   axis=1, keepdims=True)
    idx_ref[...] = idxf.astype(jnp.int32).reshape(_BR)

    pmm = jnp.dot(pb_ref[...], et_ref[...], preferred_element_type=jnp.float32)
    pos_prob_ref[...] = soft(pmm)


def _distance_softmax(z_flat, zpos_flat, et):
    n = z_flat.shape[0]
    nb = n // _BR
    grid = (nb,)
    return pl.pallas_call(
        _vq_body,
        grid=grid,
        in_specs=[
            pl.BlockSpec((_BR, _D), lambda i: (i, 0)),
            pl.BlockSpec((_BR, _D), lambda i: (i, 0)),
            pl.BlockSpec((_D, _K), lambda i: (0, 0)),
        ],
        out_specs=[
            pl.BlockSpec((_BR, _K), lambda i: (i, 0)),
            pl.BlockSpec((_BR, _K), lambda i: (i, 0)),
            pl.BlockSpec((_BR,), lambda i: (i,)),
        ],
        out_shape=[
            jax.ShapeDtypeStruct((n, _K), jnp.float32),
            jax.ShapeDtypeStruct((n, _K), jnp.float32),
            jax.ShapeDtypeStruct((n,), jnp.int32),
        ],
        scratch_shapes=[pltpu.VMEM((1, _K), jnp.float32),
                        pltpu.VMEM((1, _K), jnp.float32)],
        compiler_params=pltpu.CompilerParams(
            dimension_semantics=("arbitrary",),
        ),
    )(z_flat, zpos_flat, et)


def _sc_gather(table, idx):
    info = plsc.get_sparse_core_info()
    nc, ns = info.num_cores, info.num_subcores
    nw = nc * ns
    b = idx.shape[0]
    b_per_w = b // nw
    chunk = 96
    nch = b_per_w // chunk
    mesh = plsc.VectorSubcoreMesh(core_axis_name="c", subcore_axis_name="s")

    @functools.partial(
        pl.kernel,
        out_type=jax.ShapeDtypeStruct((b, _D), jnp.float32),
        mesh=mesh,
        scratch_types=[
            pltpu.VMEM((nch, chunk), jnp.int32),
            pltpu.VMEM((nch, chunk, _D), jnp.float32),
            pltpu.SemaphoreType.DMA,
        ],
    )
    def k(table_hbm, idx_hbm, out_hbm, idx_v, rows_v, sem):
        wid = lax.axis_index("s") * nc + lax.axis_index("c")
        base = wid * b_per_w
        for c in range(nch):
            pltpu.sync_copy(idx_hbm.at[pl.ds(base + c * chunk, chunk)],
                            idx_v.at[c])
        copies = [
            pltpu.async_copy(table_hbm.at[idx_v.at[c]], rows_v.at[c], sem)
            for c in range(nch)
        ]
        for c in range(nch):
            copies[c].wait()
            pltpu.sync_copy(rows_v.at[c],
                            out_hbm.at[pl.ds(base + c * chunk, chunk)])

    return k(table, idx)


def kernel(z, z_pos, embedding):
    b, d, h, w = z.shape
    z_flat = jnp.transpose(z, (0, 2, 3, 1)).reshape(-1, _D)
    zpos_flat = jnp.transpose(z_pos, (0, 2, 3, 1)).reshape(-1, _D)
    et = embedding.T

    prob, pos_prob, idx = _distance_softmax(z_flat, zpos_flat, et)

    zq_flat = _sc_gather(embedding, idx)
    z_q = jnp.transpose(zq_flat.reshape(b, h, w, _D), (0, 3, 1, 2))
    return (z_q, prob, pos_prob, idx)
